# baseline (speedup 1.0000x reference)
import jax
import jax.numpy as jnp
from jax import lax
from jax.experimental import pallas as pl
from jax.experimental.pallas import tpu as pltpu
from jax.experimental.pallas import tpu_sc as plsc

B = 1024
N_VERTS = 5023
N_FACES = 9976
N_LMK = 105
L = 16
NC = 2
NS = 16
NW = NC * NS
NG = 7
LPAD = NG * L
NB = 28
NU = 3 * NB
TS = 16


def _body(vt_hbm, trisf_hbm, fidx_hbm, bary_hbm, out_hbm,
          fidx_v, trisbuf, bary_v,
          w0, w1, w2, tbl,
          gbA, gbB, gbC, obA, obB, obC,
          tsem, gsemA, gsemB, gsemC, osemA, osemB, osemC):
    wid = lax.axis_index("s") * NC + lax.axis_index("c")

    iota = lax.iota(jnp.int32, L)

    for g in range(NG):
        fidx_v[pl.ds(g * L, L)] = jnp.zeros((L,), jnp.int32)
    pltpu.sync_copy(fidx_hbm, fidx_v.at[pl.ds(0, N_LMK)])
    for t in range(3 * NG):
        bary_v[pl.ds(t * L, L)] = jnp.zeros((L,), jnp.float32)
    pltpu.sync_copy(bary_hbm, bary_v.at[pl.ds(0, 3 * N_LMK)])
    for t in range(NB * TS // L):
        tbl[pl.ds(t * L, L)] = jnp.zeros((L,), jnp.int32)
    pltpu.async_copy(trisf_hbm, trisbuf, tsem).wait()

    ws = (w0, w1, w2)
    for k in range(3):
        for g in range(NG):
            lidx = g * L + iota
            fch = fidx_v[pl.ds(g * L, L)]
            vert = plsc.load_gather(trisbuf, [3 * fch + k])
            ws[k][pl.ds(g * L, L)] = plsc.load_gather(bary_v, [3 * lidx + k])
            pos = (lidx // 4) * TS + 4 * k + lax.rem(lidx, 4)
            plsc.store_scatter(tbl, [pos], vert)

    def fire_gather(u, gb, gsem):
        c = u // NB
        lb = lax.rem(u, NB)
        off = pl.multiple_of(TS * lb, 8)
        pltpu.async_copy(vt_hbm.at[c].at[tbl.at[pl.ds(off, TS)]], gb, gsem)

    def drain_gather(u, gb, gsem):
        c = u // NB
        lb = lax.rem(u, NB)
        off = pl.multiple_of(TS * lb, 8)
        pltpu.make_async_copy(vt_hbm.at[c].at[tbl.at[pl.ds(off, TS)]], gb,
                              gsem).wait()

    def compute(u, gb, ob):
        lb = lax.rem(u, NB)
        woff = 4 * lb
        wv = [ws[k][pl.ds(woff, L)] for k in range(3)]
        s0 = [wv[0][dl] for dl in range(4)]
        s1 = [wv[1][dl] for dl in range(4)]
        s2 = [wv[2][dl] for dl in range(4)]

        @plsc.parallel_loop(0, B // L, unroll=4)
        def chunk(t):
            sl = pl.ds(t * L, L)
            for dl in range(4):
                acc = gb[dl, sl] * s0[dl]
                acc += gb[4 + dl, sl] * s1[dl]
                acc += gb[8 + dl, sl] * s2[dl]
                ob[dl, sl] = acc

    def fire_out(u, ob, osem):
        c = u // NB
        lb = lax.rem(u, NB)
        pltpu.async_copy(ob, out_hbm.at[c].at[pl.ds(4 * lb, 4)], osem)

    def drain_out(u, ob, osem):
        c = u // NB
        lb = lax.rem(u, NB)
        pltpu.make_async_copy(ob, out_hbm.at[c].at[pl.ds(4 * lb, 4)],
                              osem).wait()

    uA = wid
    uB = wid + NW
    uC = wid + 2 * NW
    has_c = wid < NU - 2 * NW

    fire_gather(uA, gbA, gsemA)
    fire_gather(uB, gbB, gsemB)

    @pl.when(has_c)
    def _():
        fire_gather(uC, gbC, gsemC)

    drain_gather(uA, gbA, gsemA)
    compute(uA, gbA, obA)
    fire_out(uA, obA, osemA)

    drain_gather(uB, gbB, gsemB)
    compute(uB, gbB, obB)
    fire_out(uB, obB, osemB)

    @pl.when(has_c)
    def _():
        drain_gather(uC, gbC, gsemC)
        compute(uC, gbC, obC)
        fire_out(uC, obC, osemC)

    drain_out(uA, obA, osemA)
    drain_out(uB, obB, osemB)

    @pl.when(has_c)
    def _():
        drain_out(uC, obC, osemC)


@jax.jit
def _lmk_sc(vt, tris_flat, lmk_faces_idx, bary_flat):
    mesh = plsc.VectorSubcoreMesh(core_axis_name="c", subcore_axis_name="s",
                                  num_cores=NC, num_subcores=NS)
    return pl.kernel(
        _body,
        out_type=jax.ShapeDtypeStruct((3, LPAD, B), jnp.float32),
        mesh=mesh,
        compiler_params=pltpu.CompilerParams(needs_layout_passes=False,
                                             use_tc_tiling_on_sc=True),
        scratch_types=[
            pltpu.VMEM((LPAD,), jnp.int32),
            pltpu.VMEM((3 * N_FACES,), jnp.int32),
            pltpu.VMEM((3 * LPAD,), jnp.float32),
            pltpu.VMEM((LPAD + L,), jnp.float32),
            pltpu.VMEM((LPAD + L,), jnp.float32),
            pltpu.VMEM((LPAD + L,), jnp.float32),
            pltpu.VMEM((NB * TS,), jnp.int32),
            pltpu.VMEM((TS, B), jnp.float32),
            pltpu.VMEM((TS, B), jnp.float32),
            pltpu.VMEM((TS, B), jnp.float32),
            pltpu.VMEM((4, B), jnp.float32),
            pltpu.VMEM((4, B), jnp.float32),
            pltpu.VMEM((4, B), jnp.float32),
            pltpu.SemaphoreType.DMA,
            pltpu.SemaphoreType.DMA,
            pltpu.SemaphoreType.DMA,
            pltpu.SemaphoreType.DMA,
            pltpu.SemaphoreType.DMA,
            pltpu.SemaphoreType.DMA,
            pltpu.SemaphoreType.DMA,
        ],
    )(vt, tris_flat, lmk_faces_idx, bary_flat)


def kernel(v, poses, tris, lmk_faces_idx, lmk_bary_coords):
    del poses
    vt = jnp.transpose(v, (2, 1, 0))
    out_t = _lmk_sc(vt,
                    tris.astype(jnp.int32).reshape(3 * N_FACES),
                    lmk_faces_idx.astype(jnp.int32),
                    lmk_bary_coords.astype(jnp.float32).reshape(3 * N_LMK))
    return jnp.transpose(out_t, (2, 1, 0))[:, :N_LMK, :]


# speedup vs baseline: 1.1207x; 1.1207x over previous
import jax
import jax.numpy as jnp
from jax import lax
from jax.experimental import pallas as pl
from jax.experimental.pallas import tpu as pltpu
from jax.experimental.pallas import tpu_sc as plsc

B = 1024
N_VERTS = 5023
N_FACES = 9976
N_LMK = 105
L = 16
NC = 2
NS = 16
NW = NC * NS
NG = 7
LPAD = NG * L
NB = 14
NU = 3 * NB


def _body(vt_hbm, trisf_hbm, fidx_hbm, bary_hbm, out_hbm,
          fidx_v, trisbuf, bary_v,
          w0, w1, w2, base0, base1, base2, tbl,
          gbA, gbB, obA, obB,
          tsem, gsemA, gsemB, osemA, osemB):
    wid = lax.axis_index("s") * NC + lax.axis_index("c")

    iota = lax.iota(jnp.int32, L)

    for g in range(NG):
        fidx_v[pl.ds(g * L, L)] = jnp.zeros((L,), jnp.int32)
    pltpu.sync_copy(fidx_hbm, fidx_v.at[pl.ds(0, N_LMK)])
    for t in range(3 * NG):
        bary_v[pl.ds(t * L, L)] = jnp.zeros((L,), jnp.float32)
    pltpu.sync_copy(bary_hbm, bary_v.at[pl.ds(0, 3 * N_LMK)])
    pltpu.async_copy(trisf_hbm, trisbuf, tsem).wait()

    ws = (w0, w1, w2)
    bases = (base0, base1, base2)
    for k in range(3):
        for g in range(NG):
            lidx = g * L + iota
            fch = fidx_v[pl.ds(g * L, L)]
            bases[k][pl.ds(g * L, L)] = plsc.load_gather(trisbuf, [3 * fch + k])
            ws[k][pl.ds(g * L, L)] = plsc.load_gather(bary_v, [3 * lidx + k])
            pos = (lidx // 8) * 24 + 8 * k + lax.rem(lidx, 8)
            plsc.store_scatter(tbl, [pos], bases[k][pl.ds(g * L, L)])

    def unit(u, gb, ob, gsem, osem):
        c = u // NB
        lb = lax.rem(u, NB)
        return c, lb

    def fire_gather(u, gb, gsem):
        c = u // NB
        lb = lax.rem(u, NB)
        off = pl.multiple_of(24 * lb, 8)
        pltpu.async_copy(vt_hbm.at[c].at[tbl.at[pl.ds(off, 24)]], gb, gsem)

    def drain_gather(u, gb, gsem):
        c = u // NB
        lb = lax.rem(u, NB)
        off = pl.multiple_of(24 * lb, 8)
        pltpu.make_async_copy(vt_hbm.at[c].at[tbl.at[pl.ds(off, 24)]], gb,
                              gsem).wait()

    def compute(u, gb, ob):
        lb = lax.rem(u, NB)
        woff = pl.multiple_of(8 * lb, 8)
        wv = [ws[k][pl.ds(woff, L)] for k in range(3)]
        s0 = [wv[0][dl] for dl in range(8)]
        s1 = [wv[1][dl] for dl in range(8)]
        s2 = [wv[2][dl] for dl in range(8)]

        @plsc.parallel_loop(0, B // L, unroll=4)
        def chunk(t):
            sl = pl.ds(t * L, L)
            for dl in range(8):
                acc = gb[dl, sl] * s0[dl]
                acc += gb[8 + dl, sl] * s1[dl]
                acc += gb[16 + dl, sl] * s2[dl]
                ob[dl, sl] = acc

    def fire_out(u, ob, osem):
        c = u // NB
        lb = lax.rem(u, NB)
        off = pl.multiple_of(8 * lb, 8)
        pltpu.async_copy(ob, out_hbm.at[c].at[pl.ds(off, 8)], osem)

    def drain_out(u, ob, osem):
        c = u // NB
        lb = lax.rem(u, NB)
        off = pl.multiple_of(8 * lb, 8)
        pltpu.make_async_copy(ob, out_hbm.at[c].at[pl.ds(off, 8)],
                              osem).wait()

    uA = wid
    uB = wid + NW
    has_b = wid < NU - NW

    fire_gather(uA, gbA, gsemA)

    @pl.when(has_b)
    def _():
        fire_gather(uB, gbB, gsemB)

    drain_gather(uA, gbA, gsemA)
    compute(uA, gbA, obA)
    fire_out(uA, obA, osemA)

    @pl.when(has_b)
    def _():
        drain_gather(uB, gbB, gsemB)
        compute(uB, gbB, obB)
        fire_out(uB, obB, osemB)

    drain_out(uA, obA, osemA)

    @pl.when(has_b)
    def _():
        drain_out(uB, obB, osemB)


@jax.jit
def _lmk_sc(vt, tris_flat, lmk_faces_idx, bary_flat):
    mesh = plsc.VectorSubcoreMesh(core_axis_name="c", subcore_axis_name="s",
                                  num_cores=NC, num_subcores=NS)
    return pl.kernel(
        _body,
        out_type=jax.ShapeDtypeStruct((3, LPAD, B), jnp.float32),
        mesh=mesh,
        compiler_params=pltpu.CompilerParams(needs_layout_passes=False,
                                             use_tc_tiling_on_sc=True),
        scratch_types=[
            pltpu.VMEM((LPAD,), jnp.int32),
            pltpu.VMEM((3 * N_FACES,), jnp.int32),
            pltpu.VMEM((3 * LPAD,), jnp.float32),
            pltpu.VMEM((2 * L * 8,), jnp.float32),
            pltpu.VMEM((2 * L * 8,), jnp.float32),
            pltpu.VMEM((2 * L * 8,), jnp.float32),
            pltpu.VMEM((LPAD,), jnp.int32),
            pltpu.VMEM((LPAD,), jnp.int32),
            pltpu.VMEM((LPAD,), jnp.int32),
            pltpu.VMEM((NB * 24,), jnp.int32),
            pltpu.VMEM((24, B), jnp.float32),
            pltpu.VMEM((24, B), jnp.float32),
            pltpu.VMEM((8, B), jnp.float32),
            pltpu.VMEM((8, B), jnp.float32),
            pltpu.SemaphoreType.DMA,
            pltpu.SemaphoreType.DMA,
            pltpu.SemaphoreType.DMA,
            pltpu.SemaphoreType.DMA,
            pltpu.SemaphoreType.DMA,
        ],
    )(vt, tris_flat, lmk_faces_idx, bary_flat)


def kernel(v, poses, tris, lmk_faces_idx, lmk_bary_coords):
    del poses
    vt = jnp.transpose(v, (2, 1, 0))
    out_t = _lmk_sc(vt,
                    tris.astype(jnp.int32).reshape(3 * N_FACES),
                    lmk_faces_idx.astype(jnp.int32),
                    lmk_bary_coords.astype(jnp.float32).reshape(3 * N_LMK))
    return jnp.transpose(out_t, (2, 1, 0))[:, :N_LMK, :]


# speedup vs baseline: 1.1517x; 1.0276x over previous
import jax
import jax.numpy as jnp
from jax import lax
from jax.experimental import pallas as pl
from jax.experimental.pallas import tpu as pltpu
from jax.experimental.pallas import tpu_sc as plsc

B = 1024
N_VERTS = 5023
N_FACES = 9976
N_LMK = 105
L = 16
NC = 2
NS = 16
NW = NC * NS
NG = 7
LPAD = NG * L
NB = 14
NU = 3 * NB


def _body(vt_hbm, trisf_hbm, fidx_hbm, bary_hbm, out_hbm,
          fidx_v, widx_v, bary_v,
          w0, w1, w2, base0, base1, base2, tbl,
          gbA, gbB, obA, obB,
          tsem, gsemA, gsemB, osemA, osemB):
    wid = lax.axis_index("s") * NC + lax.axis_index("c")

    iota = lax.iota(jnp.int32, L)

    for g in range(NG):
        fidx_v[pl.ds(g * L, L)] = jnp.zeros((L,), jnp.int32)
    pltpu.sync_copy(fidx_hbm, fidx_v.at[pl.ds(0, N_LMK)])
    for t in range(3 * NG):
        bary_v[pl.ds(t * L, L)] = jnp.zeros((L,), jnp.float32)
    pltpu.sync_copy(bary_hbm, bary_v.at[pl.ds(0, 3 * N_LMK)])

    ws = (w0, w1, w2)
    bases = (base0, base1, base2)
    for k in range(3):
        for g in range(NG):
            widx_v[pl.ds(g * L, L)] = 3 * fidx_v[pl.ds(g * L, L)] + k
        pltpu.async_copy(trisf_hbm.at[widx_v], bases[k], tsem).wait()
        for g in range(NG):
            lidx = g * L + iota
            ws[k][pl.ds(g * L, L)] = plsc.load_gather(bary_v, [3 * lidx + k])
            pos = (lidx // 8) * 24 + 8 * k + lax.rem(lidx, 8)
            plsc.store_scatter(tbl, [pos], bases[k][pl.ds(g * L, L)])

    def unit(u, gb, ob, gsem, osem):
        c = u // NB
        lb = lax.rem(u, NB)
        return c, lb

    def fire_gather(u, gb, gsem):
        c = u // NB
        lb = lax.rem(u, NB)
        off = pl.multiple_of(24 * lb, 8)
        pltpu.async_copy(vt_hbm.at[c].at[tbl.at[pl.ds(off, 24)]], gb, gsem)

    def drain_gather(u, gb, gsem):
        c = u // NB
        lb = lax.rem(u, NB)
        off = pl.multiple_of(24 * lb, 8)
        pltpu.make_async_copy(vt_hbm.at[c].at[tbl.at[pl.ds(off, 24)]], gb,
                              gsem).wait()

    def compute(u, gb, ob):
        lb = lax.rem(u, NB)
        woff = pl.multiple_of(8 * lb, 8)
        wv = [ws[k][pl.ds(woff, L)] for k in range(3)]
        s0 = [wv[0][dl] for dl in range(8)]
        s1 = [wv[1][dl] for dl in range(8)]
        s2 = [wv[2][dl] for dl in range(8)]

        @plsc.parallel_loop(0, B // L, unroll=4)
        def chunk(t):
            sl = pl.ds(t * L, L)
            for dl in range(8):
                acc = gb[dl, sl] * s0[dl]
                acc += gb[8 + dl, sl] * s1[dl]
                acc += gb[16 + dl, sl] * s2[dl]
                ob[dl, sl] = acc

    def fire_out(u, ob, osem):
        c = u // NB
        lb = lax.rem(u, NB)
        off = pl.multiple_of(8 * lb, 8)
        pltpu.async_copy(ob, out_hbm.at[c].at[pl.ds(off, 8)], osem)

    def drain_out(u, ob, osem):
        c = u // NB
        lb = lax.rem(u, NB)
        off = pl.multiple_of(8 * lb, 8)
        pltpu.make_async_copy(ob, out_hbm.at[c].at[pl.ds(off, 8)],
                              osem).wait()

    uA = wid
    uB = wid + NW
    has_b = wid < NU - NW

    fire_gather(uA, gbA, gsemA)

    @pl.when(has_b)
    def _():
        fire_gather(uB, gbB, gsemB)

    drain_gather(uA, gbA, gsemA)
    compute(uA, gbA, obA)
    fire_out(uA, obA, osemA)

    @pl.when(has_b)
    def _():
        drain_gather(uB, gbB, gsemB)
        compute(uB, gbB, obB)
        fire_out(uB, obB, osemB)

    drain_out(uA, obA, osemA)

    @pl.when(has_b)
    def _():
        drain_out(uB, obB, osemB)


@jax.jit
def _lmk_sc(vt, tris_flat, lmk_faces_idx, bary_flat):
    mesh = plsc.VectorSubcoreMesh(core_axis_name="c", subcore_axis_name="s",
                                  num_cores=NC, num_subcores=NS)
    return pl.kernel(
        _body,
        out_type=jax.ShapeDtypeStruct((3, LPAD, B), jnp.float32),
        mesh=mesh,
        compiler_params=pltpu.CompilerParams(needs_layout_passes=False,
                                             use_tc_tiling_on_sc=True),
        scratch_types=[
            pltpu.VMEM((LPAD,), jnp.int32),
            pltpu.VMEM((LPAD,), jnp.int32),
            pltpu.VMEM((3 * LPAD,), jnp.float32),
            pltpu.VMEM((2 * L * 8,), jnp.float32),
            pltpu.VMEM((2 * L * 8,), jnp.float32),
            pltpu.VMEM((2 * L * 8,), jnp.float32),
            pltpu.VMEM((LPAD,), jnp.int32),
            pltpu.VMEM((LPAD,), jnp.int32),
            pltpu.VMEM((LPAD,), jnp.int32),
            pltpu.VMEM((NB * 24,), jnp.int32),
            pltpu.VMEM((24, B), jnp.float32),
            pltpu.VMEM((24, B), jnp.float32),
            pltpu.VMEM((8, B), jnp.float32),
            pltpu.VMEM((8, B), jnp.float32),
            pltpu.SemaphoreType.DMA,
            pltpu.SemaphoreType.DMA,
            pltpu.SemaphoreType.DMA,
            pltpu.SemaphoreType.DMA,
            pltpu.SemaphoreType.DMA,
        ],
    )(vt, tris_flat, lmk_faces_idx, bary_flat)


def kernel(v, poses, tris, lmk_faces_idx, lmk_bary_coords):
    del poses
    vt = jnp.transpose(v, (2, 1, 0))
    out_t = _lmk_sc(vt,
                    tris.astype(jnp.int32).reshape(3 * N_FACES),
                    lmk_faces_idx.astype(jnp.int32),
                    lmk_bary_coords.astype(jnp.float32).reshape(3 * N_LMK))
    return jnp.transpose(out_t, (2, 1, 0))[:, :N_LMK, :]


# speedup vs baseline: 1.2477x; 1.0834x over previous
import jax
import jax.numpy as jnp
from jax import lax
from jax.experimental import pallas as pl
from jax.experimental.pallas import tpu as pltpu
from jax.experimental.pallas import tpu_sc as plsc

B = 1024
N_VERTS = 5023
N_FACES = 9976
N_LMK = 105
L = 16
NC = 2
NS = 16
NW = NC * NS
LPAD = 112
NB = 14
NU = 3 * NB


def _body(vt_hbm, trisf_hbm, fidx_hbm, bary_hbm, out_hbm,
          fidx_v, bary_v, widxA, widxB, tblA, tblB,
          gbA, gbB, obA, obB,
          tsemA, tsemB, gsemA, gsemB, osemA, osemB):
    wid = lax.axis_index("s") * NC + lax.axis_index("c")

    iota = lax.iota(jnp.int32, L)

    uA = wid
    uB = wid + NW
    has_b = wid < NU - NW

    for g in range(8):
        fidx_v[pl.ds(g * L, L)] = jnp.zeros((L,), jnp.int32)
    pltpu.sync_copy(fidx_hbm, fidx_v.at[pl.ds(0, N_LMK)])

    def build_widx(u, widx):
        lb = lax.rem(u, NB)
        faces = plsc.load_gather(fidx_v, [8 * lb + lax.rem(iota, 8)])
        widx[pl.ds(0, L)] = 3 * faces + iota // 8
        widx[pl.ds(L, L)] = 3 * faces + 2

    def fire_tris(u, widx, tbl, tsem):
        pltpu.async_copy(trisf_hbm.at[widx.at[pl.ds(0, 24)]], tbl, tsem)

    def drain_tris(u, widx, tbl, tsem):
        pltpu.make_async_copy(trisf_hbm.at[widx.at[pl.ds(0, 24)]], tbl,
                              tsem).wait()

    build_widx(uA, widxA)
    fire_tris(uA, widxA, tblA, tsemA)

    @pl.when(has_b)
    def _():
        build_widx(uB, widxB)
        fire_tris(uB, widxB, tblB, tsemB)

    for t in range(24):
        bary_v[pl.ds(t * L, L)] = jnp.zeros((L,), jnp.float32)
    pltpu.sync_copy(bary_hbm, bary_v.at[pl.ds(0, 3 * N_LMK)])

    def fire_gather(u, tbl, gb, gsem):
        c = u // NB
        pltpu.async_copy(vt_hbm.at[c].at[tbl], gb, gsem)

    def drain_gather(u, tbl, gb, gsem):
        c = u // NB
        pltpu.make_async_copy(vt_hbm.at[c].at[tbl], gb, gsem).wait()

    def compute(u, gb, ob):
        lb = lax.rem(u, NB)
        wv = [plsc.load_gather(bary_v, [3 * (8 * lb + iota) + k])
              for k in range(3)]
        s0 = [wv[0][dl] for dl in range(8)]
        s1 = [wv[1][dl] for dl in range(8)]
        s2 = [wv[2][dl] for dl in range(8)]

        @plsc.parallel_loop(0, B // L, unroll=4)
        def chunk(t):
            sl = pl.ds(t * L, L)
            for dl in range(8):
                acc = gb[dl, sl] * s0[dl]
                acc += gb[8 + dl, sl] * s1[dl]
                acc += gb[16 + dl, sl] * s2[dl]
                ob[dl, sl] = acc

    def fire_out(u, ob, osem):
        c = u // NB
        lb = lax.rem(u, NB)
        off = pl.multiple_of(8 * lb, 8)
        pltpu.async_copy(ob, out_hbm.at[c].at[pl.ds(off, 8)], osem)

    def drain_out(u, ob, osem):
        c = u // NB
        lb = lax.rem(u, NB)
        off = pl.multiple_of(8 * lb, 8)
        pltpu.make_async_copy(ob, out_hbm.at[c].at[pl.ds(off, 8)],
                              osem).wait()

    drain_tris(uA, widxA, tblA, tsemA)
    fire_gather(uA, tblA, gbA, gsemA)

    @pl.when(has_b)
    def _():
        drain_tris(uB, widxB, tblB, tsemB)
        fire_gather(uB, tblB, gbB, gsemB)

    drain_gather(uA, tblA, gbA, gsemA)
    compute(uA, gbA, obA)
    fire_out(uA, obA, osemA)

    @pl.when(has_b)
    def _():
        drain_gather(uB, tblB, gbB, gsemB)
        compute(uB, gbB, obB)
        fire_out(uB, obB, osemB)

    drain_out(uA, obA, osemA)

    @pl.when(has_b)
    def _():
        drain_out(uB, obB, osemB)


@jax.jit
def _lmk_sc(vt, tris_flat, lmk_faces_idx, bary_flat):
    mesh = plsc.VectorSubcoreMesh(core_axis_name="c", subcore_axis_name="s",
                                  num_cores=NC, num_subcores=NS)
    return pl.kernel(
        _body,
        out_type=jax.ShapeDtypeStruct((3, LPAD, B), jnp.float32),
        mesh=mesh,
        compiler_params=pltpu.CompilerParams(needs_layout_passes=False,
                                             use_tc_tiling_on_sc=True),
        scratch_types=[
            pltpu.VMEM((8 * L,), jnp.int32),
            pltpu.VMEM((24 * L,), jnp.float32),
            pltpu.VMEM((2 * L,), jnp.int32),
            pltpu.VMEM((2 * L,), jnp.int32),
            pltpu.VMEM((24,), jnp.int32),
            pltpu.VMEM((24,), jnp.int32),
            pltpu.VMEM((24, B), jnp.float32),
            pltpu.VMEM((24, B), jnp.float32),
            pltpu.VMEM((8, B), jnp.float32),
            pltpu.VMEM((8, B), jnp.float32),
            pltpu.SemaphoreType.DMA,
            pltpu.SemaphoreType.DMA,
            pltpu.SemaphoreType.DMA,
            pltpu.SemaphoreType.DMA,
            pltpu.SemaphoreType.DMA,
            pltpu.SemaphoreType.DMA,
        ],
    )(vt, tris_flat, lmk_faces_idx, bary_flat)


def kernel(v, poses, tris, lmk_faces_idx, lmk_bary_coords):
    del poses
    vt = jnp.transpose(v, (2, 1, 0))
    out_t = _lmk_sc(vt,
                    tris.astype(jnp.int32).reshape(3 * N_FACES),
                    lmk_faces_idx.astype(jnp.int32),
                    lmk_bary_coords.astype(jnp.float32).reshape(3 * N_LMK))
    return jnp.transpose(out_t, (2, 1, 0))[:, :N_LMK, :]


# speedup vs baseline: 1.3231x; 1.0604x over previous
import jax
import jax.numpy as jnp
from jax import lax
from jax.experimental import pallas as pl
from jax.experimental.pallas import tpu as pltpu
from jax.experimental.pallas import tpu_sc as plsc

B = 1024
N_VERTS = 5023
N_FACES = 9976
N_LMK = 105
L = 16
NC = 2
NS = 16
NW = NC * NS
LPAD = 112
NB = 14
NU = 3 * NB


def _body(vt_hbm, trist_hbm, fidx_hbm, baryt_hbm, out_hbm,
          fidx_v, tbuf, bbuf, tblA, tblB,
          gbA, gbB, obA, obB,
          tsem, gsemA, gsemB, osemA, osemB):
    wid = lax.axis_index("s") * NC + lax.axis_index("c")

    iota = lax.iota(jnp.int32, L)

    uA = wid
    uB = wid + NW
    has_b = wid < NU - NW

    tcopy = pltpu.async_copy(trist_hbm, tbuf, tsem)
    for g in range(8):
        fidx_v[pl.ds(g * L, L)] = jnp.zeros((L,), jnp.int32)
    pltpu.sync_copy(fidx_hbm, fidx_v.at[pl.ds(0, N_LMK)])
    pltpu.sync_copy(baryt_hbm, bbuf)
    tcopy.wait()

    def build_tbl(u, tbl):
        lb = lax.rem(u, NB)
        faces = plsc.load_gather(fidx_v, [8 * lb + lax.rem(iota, 8)])
        tbl[pl.ds(0, L)] = plsc.load_gather(tbuf, [iota // 8, faces])
        tbl[pl.ds(L, L)] = plsc.load_gather(
            tbuf, [jnp.full((L,), 2, jnp.int32), faces])

    def fire_gather(u, tbl, gb, gsem):
        c = u // NB
        pltpu.async_copy(vt_hbm.at[c].at[tbl.at[pl.ds(0, 24)]], gb, gsem)

    def drain_gather(u, tbl, gb, gsem):
        c = u // NB
        pltpu.make_async_copy(vt_hbm.at[c].at[tbl.at[pl.ds(0, 24)]], gb,
                              gsem).wait()

    def compute(u, gb, ob):
        lb = lax.rem(u, NB)
        lidx = 8 * lb + iota
        wv = [plsc.load_gather(bbuf, [jnp.full((L,), k, jnp.int32), lidx])
              for k in range(3)]
        s0 = [wv[0][dl] for dl in range(8)]
        s1 = [wv[1][dl] for dl in range(8)]
        s2 = [wv[2][dl] for dl in range(8)]

        @plsc.parallel_loop(0, B // L, unroll=4)
        def chunk(t):
            sl = pl.ds(t * L, L)
            for dl in range(8):
                acc = gb[dl, sl] * s0[dl]
                acc += gb[8 + dl, sl] * s1[dl]
                acc += gb[16 + dl, sl] * s2[dl]
                ob[dl, sl] = acc

    def fire_out(u, ob, osem):
        c = u // NB
        lb = lax.rem(u, NB)
        off = pl.multiple_of(8 * lb, 8)
        pltpu.async_copy(ob, out_hbm.at[c].at[pl.ds(off, 8)], osem)

    def drain_out(u, ob, osem):
        c = u // NB
        lb = lax.rem(u, NB)
        off = pl.multiple_of(8 * lb, 8)
        pltpu.make_async_copy(ob, out_hbm.at[c].at[pl.ds(off, 8)],
                              osem).wait()

    build_tbl(uA, tblA)
    fire_gather(uA, tblA, gbA, gsemA)

    @pl.when(has_b)
    def _():
        build_tbl(uB, tblB)
        fire_gather(uB, tblB, gbB, gsemB)

    drain_gather(uA, tblA, gbA, gsemA)
    compute(uA, gbA, obA)
    fire_out(uA, obA, osemA)

    @pl.when(has_b)
    def _():
        drain_gather(uB, tblB, gbB, gsemB)
        compute(uB, gbB, obB)
        fire_out(uB, obB, osemB)

    drain_out(uA, obA, osemA)

    @pl.when(has_b)
    def _():
        drain_out(uB, obB, osemB)


@jax.jit
def _lmk_sc(vt, trist, lmk_faces_idx, baryt):
    mesh = plsc.VectorSubcoreMesh(core_axis_name="c", subcore_axis_name="s",
                                  num_cores=NC, num_subcores=NS)
    return pl.kernel(
        _body,
        out_type=jax.ShapeDtypeStruct((3, LPAD, B), jnp.float32),
        mesh=mesh,
        compiler_params=pltpu.CompilerParams(needs_layout_passes=False,
                                             use_tc_tiling_on_sc=True),
        scratch_types=[
            pltpu.VMEM((8 * L,), jnp.int32),
            pltpu.VMEM((3, N_FACES), jnp.int32),
            pltpu.VMEM((3, N_LMK), jnp.float32),
            pltpu.VMEM((2 * L,), jnp.int32),
            pltpu.VMEM((2 * L,), jnp.int32),
            pltpu.VMEM((24, B), jnp.float32),
            pltpu.VMEM((24, B), jnp.float32),
            pltpu.VMEM((8, B), jnp.float32),
            pltpu.VMEM((8, B), jnp.float32),
            pltpu.SemaphoreType.DMA,
            pltpu.SemaphoreType.DMA,
            pltpu.SemaphoreType.DMA,
            pltpu.SemaphoreType.DMA,
            pltpu.SemaphoreType.DMA,
        ],
    )(vt, trist, lmk_faces_idx, baryt)


def kernel(v, poses, tris, lmk_faces_idx, lmk_bary_coords):
    del poses
    vt = jnp.transpose(v, (2, 1, 0))
    trist = jnp.transpose(tris.astype(jnp.int32), (1, 0))
    baryt = jnp.transpose(lmk_bary_coords.astype(jnp.float32), (1, 0))
    out_t = _lmk_sc(vt, trist, lmk_faces_idx.astype(jnp.int32), baryt)
    return jnp.transpose(out_t, (2, 1, 0))[:, :N_LMK, :]
